# baseline (speedup 1.0000x reference)
import jax
import jax.numpy as jnp
from jax.experimental import pallas as pl
from jax.experimental.pallas import tpu as pltpu

_BASE_LEN = 2048
_BLK = 258


def _bulk_body(kc, vc, ko, vo):
    ko[...] = kc[...]
    vo[...] = vc[...]


def _overwrite_body(pos_ref, kv, vv, _ka, _va, ko, vo, sk, sv):
    seq = kv.shape[1]
    pos = pos_ref[0]
    ck = pltpu.make_async_copy(kv, ko.at[:, pl.ds(pos, seq)], sk)
    cv = pltpu.make_async_copy(vv, vo.at[:, pl.ds(pos, seq)], sv)
    ck.start()
    cv.start()
    ck.wait()
    cv.wait()


def kernel(k_cache, v_cache, input_pos, k_val, v_val):
    bs, seq, n_heads, head_dim = k_val.shape
    out_len = _BASE_LEN + seq
    pos = jnp.asarray(input_pos, dtype=jnp.int32).reshape(1)
    out_sd = jax.ShapeDtypeStruct((bs, out_len, n_heads, head_dim), k_cache.dtype)

    n_blk = out_len // _BLK
    assert n_blk * _BLK == out_len
    blk = (1, _BLK, n_heads, head_dim)
    k_bulk, v_bulk = pl.pallas_call(
        _bulk_body,
        grid=(bs, n_blk),
        out_shape=(out_sd, out_sd),
        in_specs=[
            pl.BlockSpec(blk, lambda b, i: (b, i, 0, 0)),
            pl.BlockSpec(blk, lambda b, i: (b, i, 0, 0)),
        ],
        out_specs=(
            pl.BlockSpec(blk, lambda b, i: (b, i, 0, 0)),
            pl.BlockSpec(blk, lambda b, i: (b, i, 0, 0)),
        ),
    )(k_cache, v_cache)

    k_out, v_out = pl.pallas_call(
        _overwrite_body,
        out_shape=(out_sd, out_sd),
        in_specs=[
            pl.BlockSpec(memory_space=pltpu.SMEM),
            pl.BlockSpec(memory_space=pl.ANY),
            pl.BlockSpec(memory_space=pl.ANY),
            pl.BlockSpec(memory_space=pl.ANY),
            pl.BlockSpec(memory_space=pl.ANY),
        ],
        out_specs=(
            pl.BlockSpec(memory_space=pl.ANY),
            pl.BlockSpec(memory_space=pl.ANY),
        ),
        scratch_shapes=[pltpu.SemaphoreType.DMA] * 2,
        input_output_aliases={3: 0, 4: 1},
    )(pos, k_val, v_val, k_bulk, v_bulk)
    return (k_out, v_out)


# speedup vs baseline: 1.0732x; 1.0732x over previous
import jax
import jax.numpy as jnp
from jax.experimental import pallas as pl
from jax.experimental.pallas import tpu as pltpu

_BASE_LEN = 2048
_BLK = 688


def _bulk_body(kc, vc, ko, vo):
    ko[...] = kc[...]
    vo[...] = vc[...]


def _overwrite_body(pos_ref, kv, vv, _ka, _va, ko, vo, sk, sv):
    seq = kv.shape[1]
    pos = pos_ref[0]
    ck = pltpu.make_async_copy(kv, ko.at[:, pl.ds(pos, seq)], sk)
    cv = pltpu.make_async_copy(vv, vo.at[:, pl.ds(pos, seq)], sv)
    ck.start()
    cv.start()
    ck.wait()
    cv.wait()


def kernel(k_cache, v_cache, input_pos, k_val, v_val):
    bs, seq, n_heads, head_dim = k_val.shape
    out_len = _BASE_LEN + seq
    pos = jnp.asarray(input_pos, dtype=jnp.int32).reshape(1)
    out_sd = jax.ShapeDtypeStruct((bs, out_len, n_heads, head_dim), k_cache.dtype)

    n_blk = out_len // _BLK
    assert n_blk * _BLK == out_len
    blk = (1, _BLK, n_heads, head_dim)
    k_bulk, v_bulk = pl.pallas_call(
        _bulk_body,
        grid=(bs, n_blk),
        out_shape=(out_sd, out_sd),
        in_specs=[
            pl.BlockSpec(blk, lambda b, i: (b, i, 0, 0)),
            pl.BlockSpec(blk, lambda b, i: (b, i, 0, 0)),
        ],
        out_specs=(
            pl.BlockSpec(blk, lambda b, i: (b, i, 0, 0)),
            pl.BlockSpec(blk, lambda b, i: (b, i, 0, 0)),
        ),
    )(k_cache, v_cache)

    k_out, v_out = pl.pallas_call(
        _overwrite_body,
        out_shape=(out_sd, out_sd),
        in_specs=[
            pl.BlockSpec(memory_space=pltpu.SMEM),
            pl.BlockSpec(memory_space=pl.ANY),
            pl.BlockSpec(memory_space=pl.ANY),
            pl.BlockSpec(memory_space=pl.ANY),
            pl.BlockSpec(memory_space=pl.ANY),
        ],
        out_specs=(
            pl.BlockSpec(memory_space=pl.ANY),
            pl.BlockSpec(memory_space=pl.ANY),
        ),
        scratch_shapes=[pltpu.SemaphoreType.DMA] * 2,
        input_output_aliases={3: 0, 4: 1},
    )(pos, k_val, v_val, k_bulk, v_bulk)
    return (k_out, v_out)


# speedup vs baseline: 1.0786x; 1.0051x over previous
import jax
import jax.numpy as jnp
from jax.experimental import pallas as pl
from jax.experimental.pallas import tpu as pltpu

_BASE_LEN = 2048
_BLK = 1032


def _bulk_body(kc, vc, ko, vo):
    ko[...] = kc[...]
    vo[...] = vc[...]


def _overwrite_body(pos_ref, kv, vv, _ka, _va, ko, vo, sk, sv):
    seq = kv.shape[1]
    pos = pos_ref[0]
    ck = pltpu.make_async_copy(kv, ko.at[:, pl.ds(pos, seq)], sk)
    cv = pltpu.make_async_copy(vv, vo.at[:, pl.ds(pos, seq)], sv)
    ck.start()
    cv.start()
    ck.wait()
    cv.wait()


def kernel(k_cache, v_cache, input_pos, k_val, v_val):
    bs, seq, n_heads, head_dim = k_val.shape
    out_len = _BASE_LEN + seq
    pos = jnp.asarray(input_pos, dtype=jnp.int32).reshape(1)
    out_sd = jax.ShapeDtypeStruct((bs, out_len, n_heads, head_dim), k_cache.dtype)

    n_blk = out_len // _BLK
    assert n_blk * _BLK == out_len
    blk = (1, _BLK, n_heads, head_dim)
    k_bulk, v_bulk = pl.pallas_call(
        _bulk_body,
        grid=(bs, n_blk),
        out_shape=(out_sd, out_sd),
        in_specs=[
            pl.BlockSpec(blk, lambda b, i: (b, i, 0, 0)),
            pl.BlockSpec(blk, lambda b, i: (b, i, 0, 0)),
        ],
        out_specs=(
            pl.BlockSpec(blk, lambda b, i: (b, i, 0, 0)),
            pl.BlockSpec(blk, lambda b, i: (b, i, 0, 0)),
        ),
    )(k_cache, v_cache)

    k_out, v_out = pl.pallas_call(
        _overwrite_body,
        out_shape=(out_sd, out_sd),
        in_specs=[
            pl.BlockSpec(memory_space=pltpu.SMEM),
            pl.BlockSpec(memory_space=pl.ANY),
            pl.BlockSpec(memory_space=pl.ANY),
            pl.BlockSpec(memory_space=pl.ANY),
            pl.BlockSpec(memory_space=pl.ANY),
        ],
        out_specs=(
            pl.BlockSpec(memory_space=pl.ANY),
            pl.BlockSpec(memory_space=pl.ANY),
        ),
        scratch_shapes=[pltpu.SemaphoreType.DMA] * 2,
        input_output_aliases={3: 0, 4: 1},
    )(pos, k_val, v_val, k_bulk, v_bulk)
    return (k_out, v_out)
